# baseline (speedup 1.0000x reference)
import functools

import jax
import jax.numpy as jnp
import numpy as np
from jax.experimental import pallas as pl
from jax.experimental.pallas import tpu as pltpu

B, C, GD, GH, GW = 8, 12, 8, 16, 16
H, W = 512, 512
HB = 8
NJ = H // HB


def _interp_matrix(npix, ncell):
    w = np.arange(npix)
    g = (w + 0.5) * ncell / npix - 0.5
    f = np.floor(g).astype(np.int64)
    w1 = (g - f).astype(np.float32)
    w0 = 1.0 - w1
    m = np.zeros((ncell, npix), np.float32)
    np.add.at(m, (np.clip(f, 0, ncell - 1), w), w0)
    np.add.at(m, (np.clip(f + 1, 0, ncell - 1), w), w1)
    return m


def _body(g5_ref, rxt_ref, guide_ref, out_ref, u_ref):
    j = pl.program_id(1)

    @pl.when(j == 0)
    def _():
        u = jnp.dot(g5_ref[0], rxt_ref[...],
                    preferred_element_type=jnp.float32)
        u_ref[...] = u.reshape(GH, C, GD, W)

    fy = ((j * HB) // 16 - 1) // 2
    yi0 = jnp.clip(fy, 0, GH - 1)
    yi1 = jnp.clip(fy + 1, 0, GH - 1)
    hrow = (jax.lax.broadcasted_iota(jnp.int32, (HB, 1), 0)
            + j * HB).astype(jnp.float32)
    gy = (hrow + 0.5) * (GH / H) - 0.5
    wy1 = gy - fy.astype(jnp.float32)
    wy0 = 1.0 - wy1

    u0 = u_ref[yi0]
    u1 = u_ref[yi1]

    t = jnp.clip(guide_ref[0] * GD - 0.5, 0.0, GD - 1.0)
    wz = [jnp.maximum(1.0 - jnp.abs(t - k), 0.0) for k in range(GD)]

    for c in range(C):
        a0 = wz[0] * u0[c, 0][None, :]
        a1 = wz[0] * u1[c, 0][None, :]
        for k in range(1, GD):
            a0 = a0 + wz[k] * u0[c, k][None, :]
            a1 = a1 + wz[k] * u1[c, k][None, :]
        out_ref[0, c] = wy0 * a0 + wy1 * a1


@jax.jit
def kernel(bilateral_grid, guidemap):
    g5 = jnp.transpose(bilateral_grid, (0, 3, 1, 2, 4)).reshape(B, GH * C * GD, GW)
    rxt = jnp.asarray(_interp_matrix(W, GW))
    guide = guidemap.reshape(B, H, W)

    return pl.pallas_call(
        _body,
        grid=(B, NJ),
        in_specs=[
            pl.BlockSpec((1, GH * C * GD, GW), lambda b, j: (b, 0, 0)),
            pl.BlockSpec((GW, W), lambda b, j: (0, 0)),
            pl.BlockSpec((1, HB, W), lambda b, j: (b, j, 0)),
        ],
        out_specs=pl.BlockSpec((1, C, HB, W), lambda b, j: (b, 0, j, 0)),
        out_shape=jax.ShapeDtypeStruct((B, C, H, W), jnp.float32),
        scratch_shapes=[pltpu.VMEM((GH, C, GD, W), jnp.float32)],
    )(g5, rxt, guide)


# speedup vs baseline: 2.2457x; 2.2457x over previous
import jax
import jax.numpy as jnp
import numpy as np
from jax.experimental import pallas as pl
from jax.experimental.pallas import tpu as pltpu

B, C, GD, GH, GW = 8, 12, 8, 16, 16
H, W = 512, 512
HB = 64
SB = 16
NS = HB // SB
NJ = H // HB


def _interp_matrix(npix, ncell):
    w = np.arange(npix)
    g = (w + 0.5) * ncell / npix - 0.5
    f = np.floor(g).astype(np.int64)
    w1 = (g - f).astype(np.float32)
    w0 = 1.0 - w1
    m = np.zeros((ncell, npix), np.float32)
    np.add.at(m, (np.clip(f, 0, ncell - 1), w), w0)
    np.add.at(m, (np.clip(f + 1, 0, ncell - 1), w), w1)
    return m


def _body(g5_ref, rxt_ref, guide_ref, out_ref, u_ref):
    j = pl.program_id(1)

    @pl.when(j == 0)
    def _():
        u = jnp.dot(g5_ref[0], rxt_ref[...],
                    preferred_element_type=jnp.float32)
        u_ref[...] = u.reshape(GH, C, GD, W)

    for s in range(NS):
        band = j * NS + s
        fy = (band - 1) // 2
        yi0 = jnp.clip(fy, 0, GH - 1)
        yi1 = jnp.clip(fy + 1, 0, GH - 1)
        hrow = (jax.lax.broadcasted_iota(jnp.int32, (SB, 1), 0)
                + band * SB).astype(jnp.float32)
        gy = (hrow + 0.5) * (GH / H) - 0.5
        wy1 = gy - fy.astype(jnp.float32)
        wy0 = 1.0 - wy1

        u0 = u_ref[yi0]
        u1 = u_ref[yi1]

        g = guide_ref[0, s * SB:(s + 1) * SB]
        t = jnp.clip(g * GD - 0.5, 0.0, GD - 1.0)
        wz = [jnp.maximum(1.0 - jnp.abs(t - k), 0.0) for k in range(GD)]

        for c in range(C):
            a0 = wz[0] * u0[c, 0][None, :]
            a1 = wz[0] * u1[c, 0][None, :]
            for k in range(1, GD):
                a0 = a0 + wz[k] * u0[c, k][None, :]
                a1 = a1 + wz[k] * u1[c, k][None, :]
            out_ref[0, c, s * SB:(s + 1) * SB] = wy0 * a0 + wy1 * a1


@jax.jit
def kernel(bilateral_grid, guidemap):
    g5 = jnp.transpose(bilateral_grid, (0, 3, 1, 2, 4)).reshape(B, GH * C * GD, GW)
    rxt = jnp.asarray(_interp_matrix(W, GW))
    guide = guidemap.reshape(B, H, W)

    return pl.pallas_call(
        _body,
        grid=(B, NJ),
        in_specs=[
            pl.BlockSpec((1, GH * C * GD, GW), lambda b, j: (b, 0, 0)),
            pl.BlockSpec((GW, W), lambda b, j: (0, 0)),
            pl.BlockSpec((1, HB, W), lambda b, j: (b, j, 0)),
        ],
        out_specs=pl.BlockSpec((1, C, HB, W), lambda b, j: (b, 0, j, 0)),
        out_shape=jax.ShapeDtypeStruct((B, C, H, W), jnp.float32),
        scratch_shapes=[pltpu.VMEM((GH, C, GD, W), jnp.float32)],
    )(g5, rxt, guide)


# speedup vs baseline: 2.2530x; 1.0032x over previous
import jax
import jax.numpy as jnp
import numpy as np
from jax.experimental import pallas as pl
from jax.experimental.pallas import tpu as pltpu

B, C, GD, GH, GW = 8, 12, 8, 16, 16
H, W = 512, 512
HB = 128
SB = 16
NS = HB // SB
NJ = H // HB


def _interp_matrix(npix, ncell):
    w = np.arange(npix)
    g = (w + 0.5) * ncell / npix - 0.5
    f = np.floor(g).astype(np.int64)
    w1 = (g - f).astype(np.float32)
    w0 = 1.0 - w1
    m = np.zeros((ncell, npix), np.float32)
    np.add.at(m, (np.clip(f, 0, ncell - 1), w), w0)
    np.add.at(m, (np.clip(f + 1, 0, ncell - 1), w), w1)
    return m


def _body(g5_ref, rxt_ref, guide_ref, out_ref, u_ref):
    j = pl.program_id(1)

    @pl.when(j == 0)
    def _():
        u = jnp.dot(g5_ref[0], rxt_ref[...],
                    preferred_element_type=jnp.float32)
        u_ref[...] = u.reshape(GH, C, GD, W)

    for s in range(NS):
        band = j * NS + s
        fy = (band - 1) // 2
        yi0 = jnp.clip(fy, 0, GH - 1)
        yi1 = jnp.clip(fy + 1, 0, GH - 1)
        hrow = (jax.lax.broadcasted_iota(jnp.int32, (SB, 1), 0)
                + band * SB).astype(jnp.float32)
        gy = (hrow + 0.5) * (GH / H) - 0.5
        wy1 = gy - fy.astype(jnp.float32)
        wy0 = 1.0 - wy1

        u0 = u_ref[yi0]
        u1 = u_ref[yi1]

        g = guide_ref[0, s * SB:(s + 1) * SB]
        t = jnp.clip(g * GD - 0.5, 0.0, GD - 1.0)
        wz = [jnp.maximum(1.0 - jnp.abs(t - k), 0.0) for k in range(GD)]

        for c in range(C):
            a0 = wz[0] * u0[c, 0][None, :]
            a1 = wz[0] * u1[c, 0][None, :]
            for k in range(1, GD):
                a0 = a0 + wz[k] * u0[c, k][None, :]
                a1 = a1 + wz[k] * u1[c, k][None, :]
            out_ref[0, c, s * SB:(s + 1) * SB] = wy0 * a0 + wy1 * a1


@jax.jit
def kernel(bilateral_grid, guidemap):
    g5 = jnp.transpose(bilateral_grid, (0, 3, 1, 2, 4)).reshape(B, GH * C * GD, GW)
    rxt = jnp.asarray(_interp_matrix(W, GW))
    guide = guidemap.reshape(B, H, W)

    return pl.pallas_call(
        _body,
        grid=(B, NJ),
        in_specs=[
            pl.BlockSpec((1, GH * C * GD, GW), lambda b, j: (b, 0, 0)),
            pl.BlockSpec((GW, W), lambda b, j: (0, 0)),
            pl.BlockSpec((1, HB, W), lambda b, j: (b, j, 0)),
        ],
        out_specs=pl.BlockSpec((1, C, HB, W), lambda b, j: (b, 0, j, 0)),
        out_shape=jax.ShapeDtypeStruct((B, C, H, W), jnp.float32),
        scratch_shapes=[pltpu.VMEM((GH, C, GD, W), jnp.float32)],
    )(g5, rxt, guide)
